# baseline (speedup 1.0000x reference)
import functools

import jax
import jax.numpy as jnp
from jax import lax
from jax.experimental import pallas as pl
from jax.experimental.pallas import tpu as pltpu
from jax.experimental.pallas import tpu_sc as plsc

B = 4096
L = 200
D = 64
NC = 2
NS = 16
NW = NC * NS
BPW = B // NW
C0 = 104
C1 = L - C0


def _pool_call(fiu, fii, untf, intf, user_emb_W, item_emb_W):
  mesh = plsc.VectorSubcoreMesh(core_axis_name="c", subcore_axis_name="s")
  out_t = (jax.ShapeDtypeStruct((B, D), jnp.float32),
           jax.ShapeDtypeStruct((B, D), jnp.float32))

  @functools.partial(
      pl.kernel, mesh=mesh, out_type=out_t,
      compiler_params=pltpu.CompilerParams(use_tc_tiling_on_sc=False),
      scratch_types=[
          pltpu.VMEM((BPW * 32,), jnp.int32),
          pltpu.VMEM((BPW * 32,), jnp.int32),
          pltpu.VMEM((BPW * 32, 8), jnp.int32),
          pltpu.VMEM((BPW * 32, 8), jnp.int32),
          pltpu.VMEM((L, D), jnp.float32),
          pltpu.VMEM((L, D), jnp.float32),
          pltpu.VMEM((BPW, D), jnp.float32),
          pltpu.VMEM((BPW, D), jnp.float32),
          pltpu.SemaphoreType.DMA,
          pltpu.SemaphoreType.DMA,
          pltpu.SemaphoreType.DMA,
      ])
  def pool(fiu_hbm, fii_hbm, unt_hbm, int_hbm, uemb_hbm, iemb_hbm,
           uout_hbm, iout_hbm,
           fiu_v, fii_v, uneigh_v, ineigh_v, buf_a, buf_b, uout_v, iout_v,
           sem_a, sem_b, sem_n):
    wid = lax.axis_index("s") * NC + lax.axis_index("c")
    base = wid * BPW
    pltpu.sync_copy(fiu_hbm.at[wid], fiu_v)
    pltpu.sync_copy(fii_hbm.at[wid], fii_v)

    @pl.loop(0, 32)
    def _(k):
      pltpu.async_copy(unt_hbm.at[fiu_v.at[pl.ds(128 * k, 128)]],
                       uneigh_v.at[pl.ds(128 * k, 128)], sem_n)
      pltpu.async_copy(int_hbm.at[fii_v.at[pl.ds(128 * k, 128)]],
                       ineigh_v.at[pl.ds(128 * k, 128)], sem_n)

    @pl.loop(0, 32)
    def _(k):
      pltpu.make_async_copy(unt_hbm.at[pl.ds(0, 128)],
                            uneigh_v.at[pl.ds(0, 128)], sem_n).wait()
      pltpu.make_async_copy(unt_hbm.at[pl.ds(0, 128)],
                            ineigh_v.at[pl.ds(0, 128)], sem_n).wait()

    def fire(emb_hbm, neigh_v, i, buf, sem):
      @pl.loop(0, L // 8)
      def _(r):
        pltpu.async_copy(emb_hbm.at[neigh_v.at[32 * i + r]],
                         buf.at[pl.ds(8 * r, 8)], sem)

    def drain(emb_hbm, buf, sem):
      pltpu.make_async_copy(emb_hbm.at[pl.ds(0, L)], buf, sem).wait()

    def reduce_into(buf, out_v, i):
      def body(r, accs):
        return tuple(accs[c] + buf[r, pl.ds(16 * c, 16)] for c in range(4))
      z = jnp.zeros((16,), jnp.float32)
      accs = lax.fori_loop(0, L, body, (z, z, z, z), unroll=8)
      for c in range(4):
        out_v[i, pl.ds(16 * c, 16)] = accs[c]

    def do_side(emb_hbm, neigh_v, out_v):
      fire(emb_hbm, neigh_v, 0, buf_a, sem_a)

      @pl.loop(0, BPW, step=2)
      def _(i):
        fire(emb_hbm, neigh_v, i + 1, buf_b, sem_b)
        drain(emb_hbm, buf_a, sem_a)
        reduce_into(buf_a, out_v, i)

        @pl.when(i + 2 < BPW)
        def _():
          fire(emb_hbm, neigh_v, i + 2, buf_a, sem_a)

        drain(emb_hbm, buf_b, sem_b)
        reduce_into(buf_b, out_v, i + 1)

    do_side(uemb_hbm, uneigh_v, uout_v)
    do_side(iemb_hbm, ineigh_v, iout_v)
    pltpu.sync_copy(uout_v, uout_hbm.at[pl.ds(base, BPW)])
    pltpu.sync_copy(iout_v, iout_hbm.at[pl.ds(base, BPW)])

  return pool(fiu, fii, untf, intf, user_emb_W, item_emb_W)


def _mlp_call(u_sum, i_sum, w1, b1, w2, b2, w3, b3, w4, b4, w5, b5):
  n = B * D
  rows = 4096
  grid = n // rows
  elems = rows // D
  u = u_sum.reshape(n, 1)
  v = i_sum.reshape(n, 1)
  s_avg = jnp.kron(jnp.eye(elems, dtype=jnp.float32),
                   jnp.full((1, D), 1.0 / D, jnp.float32))
  b1r, b2r, b3r, b4r, b5r = (bb.reshape(1, -1) for bb in (b1, b2, b3, b4, b5))

  def body(u_ref, v_ref, w1_ref, b1_ref, w2_ref, b2_ref, w3_ref, b3_ref,
           w4_ref, b4_ref, w5_ref, b5_ref, s_ref, o_ref):
    w1v = w1_ref[...]
    h = jnp.maximum(
        u_ref[...] * w1v[0:1, :] + v_ref[...] * w1v[1:2, :] + b1_ref[...], 0.0)
    h = jnp.maximum(
        jnp.dot(h, w2_ref[...], preferred_element_type=jnp.float32)
        + b2_ref[...], 0.0)
    h = jnp.maximum(
        jnp.dot(h, w3_ref[...], preferred_element_type=jnp.float32)
        + b3_ref[...], 0.0)
    h = jnp.maximum(
        jnp.dot(h, w4_ref[...], preferred_element_type=jnp.float32)
        + b4_ref[...], 0.0)
    o = jnp.dot(h, w5_ref[...], preferred_element_type=jnp.float32) + b5_ref[...]
    m = jnp.dot(s_ref[...], o, preferred_element_type=jnp.float32)
    o_ref[...] = jax.nn.sigmoid(m)

  def full(a):
    nd = a.ndim
    return pl.BlockSpec(a.shape, lambda g, _nd=nd: (0,) * _nd)

  out = pl.pallas_call(
      body,
      grid=(grid,),
      in_specs=[
          pl.BlockSpec((rows, 1), lambda g: (g, 0)),
          pl.BlockSpec((rows, 1), lambda g: (g, 0)),
          full(w1), full(b1r), full(w2), full(b2r), full(w3), full(b3r),
          full(w4), full(b4r), full(w5), full(b5r), full(s_avg),
      ],
      out_specs=pl.BlockSpec((elems, 1), lambda g: (g, 0)),
      out_shape=jax.ShapeDtypeStruct((B, 1), jnp.float32),
  )(u, v, w1, b1r, w2, b2r, w3, b3r, w4, b4r, w5, b5r, s_avg)
  return out.reshape(B)


def kernel(user_idxs, item_idxs, user_idx_tensor, item_idx_tensor,
           user_emb_W, item_emb_W, w1, b1, w2, b2, w3, b3, w4, b4, w5, b5):
  untf = user_idx_tensor.reshape(-1, 8)
  intf = item_idx_tensor.reshape(-1, 8)
  ks = jnp.minimum(jnp.arange(32, dtype=jnp.int32), L // 8 - 1)
  fiu = (user_idxs.astype(jnp.int32)[:, None] * (L // 8)
         + ks[None, :]).reshape(NW, BPW * 32)
  fii = (item_idxs.astype(jnp.int32)[:, None] * (L // 8)
         + ks[None, :]).reshape(NW, BPW * 32)
  u_sum, i_sum = _pool_call(fiu, fii, untf, intf, user_emb_W, item_emb_W)
  return _mlp_call(u_sum, i_sum, w1, b1, w2, b2, w3, b3, w4, b4, w5, b5)


# speedup vs baseline: 1.0854x; 1.0854x over previous
import functools

import jax
import jax.numpy as jnp
from jax import lax
from jax.experimental import pallas as pl
from jax.experimental.pallas import tpu as pltpu
from jax.experimental.pallas import tpu_sc as plsc

B = 4096
L = 200
D = 64
NC = 2
NS = 16
NW = NC * NS
BPW = B // NW
C0 = 104
C1 = L - C0


def _pool_call(uneigh, ineigh, user_emb_W, item_emb_W):
  mesh = plsc.VectorSubcoreMesh(core_axis_name="c", subcore_axis_name="s")
  out_t = (jax.ShapeDtypeStruct((B, D), jnp.float32),
           jax.ShapeDtypeStruct((B, D), jnp.float32))

  @functools.partial(
      pl.kernel, mesh=mesh, out_type=out_t,
      compiler_params=pltpu.CompilerParams(use_tc_tiling_on_sc=False),
      scratch_types=[
          pltpu.VMEM((BPW, L), jnp.int32),
          pltpu.VMEM((BPW, L), jnp.int32),
          pltpu.VMEM((L, D), jnp.float32),
          pltpu.VMEM((L, D), jnp.float32),
          pltpu.VMEM((BPW, D), jnp.float32),
          pltpu.VMEM((BPW, D), jnp.float32),
          pltpu.SemaphoreType.DMA,
          pltpu.SemaphoreType.DMA,
          pltpu.SemaphoreType.DMA,
      ])
  def pool(un_hbm, in_hbm, uemb_hbm, iemb_hbm,
           uout_hbm, iout_hbm,
           uneigh_v, ineigh_v, buf_a, buf_b, uout_v, iout_v,
           sem_a, sem_b, sem_n):
    wid = lax.axis_index("s") * NC + lax.axis_index("c")
    base = wid * BPW
    pltpu.async_copy(un_hbm.at[pl.ds(base, BPW)], uneigh_v, sem_n)
    pltpu.async_copy(in_hbm.at[pl.ds(base, BPW)], ineigh_v, sem_n)
    pltpu.make_async_copy(un_hbm.at[pl.ds(0, BPW)], uneigh_v, sem_n).wait()
    pltpu.make_async_copy(in_hbm.at[pl.ds(0, BPW)], ineigh_v, sem_n).wait()

    def fire(emb_hbm, neigh_v, i, buf, sem):
      pltpu.async_copy(emb_hbm.at[neigh_v.at[i, pl.ds(0, C0)]],
                       buf.at[pl.ds(0, C0)], sem)
      pltpu.async_copy(emb_hbm.at[neigh_v.at[i, pl.ds(C0, C1)]],
                       buf.at[pl.ds(C0, C1)], sem)

    def drain(emb_hbm, buf, sem):
      pltpu.make_async_copy(emb_hbm.at[pl.ds(0, L)], buf, sem).wait()

    def reduce_into(buf, out_v, i):
      def body(r, accs):
        return tuple(accs[c] + buf[r, pl.ds(16 * c, 16)] for c in range(4))
      z = jnp.zeros((16,), jnp.float32)
      accs = lax.fori_loop(0, L, body, (z, z, z, z), unroll=8)
      for c in range(4):
        out_v[i, pl.ds(16 * c, 16)] = accs[c]

    def do_side(emb_hbm, neigh_v, out_v):
      fire(emb_hbm, neigh_v, 0, buf_a, sem_a)

      @pl.loop(0, BPW, step=2)
      def _(i):
        fire(emb_hbm, neigh_v, i + 1, buf_b, sem_b)
        drain(emb_hbm, buf_a, sem_a)
        reduce_into(buf_a, out_v, i)

        @pl.when(i + 2 < BPW)
        def _():
          fire(emb_hbm, neigh_v, i + 2, buf_a, sem_a)

        drain(emb_hbm, buf_b, sem_b)
        reduce_into(buf_b, out_v, i + 1)

    do_side(uemb_hbm, uneigh_v, uout_v)
    do_side(iemb_hbm, ineigh_v, iout_v)
    pltpu.sync_copy(uout_v, uout_hbm.at[pl.ds(base, BPW)])
    pltpu.sync_copy(iout_v, iout_hbm.at[pl.ds(base, BPW)])

  return pool(uneigh, ineigh, user_emb_W, item_emb_W)


def _mlp_call(u_sum, i_sum, w1, b1, w2, b2, w3, b3, w4, b4, w5, b5):
  n = B * D
  rows = 4096
  grid = n // rows
  elems = rows // D
  u = u_sum.reshape(n, 1)
  v = i_sum.reshape(n, 1)
  s_avg = jnp.kron(jnp.eye(elems, dtype=jnp.float32),
                   jnp.full((1, D), 1.0 / D, jnp.float32))
  b1r, b2r, b3r, b4r, b5r = (bb.reshape(1, -1) for bb in (b1, b2, b3, b4, b5))

  def body(u_ref, v_ref, w1_ref, b1_ref, w2_ref, b2_ref, w3_ref, b3_ref,
           w4_ref, b4_ref, w5_ref, b5_ref, s_ref, o_ref):
    w1v = w1_ref[...]
    h = jnp.maximum(
        u_ref[...] * w1v[0:1, :] + v_ref[...] * w1v[1:2, :] + b1_ref[...], 0.0)
    h = jnp.maximum(
        jnp.dot(h, w2_ref[...], preferred_element_type=jnp.float32)
        + b2_ref[...], 0.0)
    h = jnp.maximum(
        jnp.dot(h, w3_ref[...], preferred_element_type=jnp.float32)
        + b3_ref[...], 0.0)
    h = jnp.maximum(
        jnp.dot(h, w4_ref[...], preferred_element_type=jnp.float32)
        + b4_ref[...], 0.0)
    o = jnp.dot(h, w5_ref[...], preferred_element_type=jnp.float32) + b5_ref[...]
    m = jnp.dot(s_ref[...], o, preferred_element_type=jnp.float32)
    o_ref[...] = jax.nn.sigmoid(m)

  def full(a):
    nd = a.ndim
    return pl.BlockSpec(a.shape, lambda g, _nd=nd: (0,) * _nd)

  out = pl.pallas_call(
      body,
      grid=(grid,),
      in_specs=[
          pl.BlockSpec((rows, 1), lambda g: (g, 0)),
          pl.BlockSpec((rows, 1), lambda g: (g, 0)),
          full(w1), full(b1r), full(w2), full(b2r), full(w3), full(b3r),
          full(w4), full(b4r), full(w5), full(b5r), full(s_avg),
      ],
      out_specs=pl.BlockSpec((elems, 1), lambda g: (g, 0)),
      out_shape=jax.ShapeDtypeStruct((B, 1), jnp.float32),
  )(u, v, w1, b1r, w2, b2r, w3, b3r, w4, b4r, w5, b5r, s_avg)
  return out.reshape(B)


def kernel(user_idxs, item_idxs, user_idx_tensor, item_idx_tensor,
           user_emb_W, item_emb_W, w1, b1, w2, b2, w3, b3, w4, b4, w5, b5):
  uneigh = jnp.take(user_idx_tensor, user_idxs, axis=0).astype(jnp.int32)
  ineigh = jnp.take(item_idx_tensor, item_idxs, axis=0).astype(jnp.int32)
  u_sum, i_sum = _pool_call(uneigh, ineigh, user_emb_W, item_emb_W)
  return _mlp_call(u_sum, i_sum, w1, b1, w2, b2, w3, b3, w4, b4, w5, b5)
